# baseline (speedup 1.0000x reference)
import functools

import jax
import jax.numpy as jnp
from jax import lax
from jax.experimental import pallas as pl
from jax.experimental.pallas import tpu as pltpu
from jax.experimental.pallas import tpu_sc as plsc

B = 4096
EMB = 64
LANES = 128
BROWS = 64
BCOLS = 15625

_info = plsc.get_sparse_core_info()
_NC, _NS = _info.num_cores, _info.num_subcores
_NW = _NC * _NS
_BPW = B // _NW

_G = 4
_NB = 2 * _G
_NT = _BPW // _G

_sc_mesh = plsc.VectorSubcoreMesh(core_axis_name="c", subcore_axis_name="s")


@functools.partial(
    pl.kernel,
    mesh=_sc_mesh,
    out_type=[
        jax.ShapeDtypeStruct((B, EMB), jnp.float32),
        jax.ShapeDtypeStruct((B,), jnp.float32),
    ],
    scratch_types=[
        pltpu.VMEM((_BPW,), jnp.int32),
        pltpu.VMEM((_NB, EMB, LANES), jnp.float32),
        pltpu.VMEM((_NB, 1, LANES), jnp.float32),
        pltpu.VMEM((_BPW, EMB), jnp.float32),
        pltpu.VMEM((_BPW,), jnp.float32),
        pltpu.SemaphoreType.DMA,
        pltpu.SemaphoreType.DMA,
    ],
    compiler_params=pltpu.CompilerParams(needs_layout_passes=False),
)
def _sc_gather(embT_hbm, bias2_hbm, idx_hbm, w_out, b_out,
               idx_v, bufs, bbufs, w_v, bias_v, sem_a, sem_b):
    wid = lax.axis_index("s") * _NC + lax.axis_index("c")
    base = wid * _BPW
    pltpu.sync_copy(idx_hbm.at[pl.ds(base, _BPW)], idx_v)
    sems = (sem_a, sem_b)
    iota16 = jax.lax.iota(jnp.int32, 16)

    def fire_batch(t):
        g = t % 2
        vec = idx_v[pl.ds((t * _G // 16) * 16, 16)]
        for i in range(_G):
            k = t * _G + i
            r = vec[k % 16]
            off = pl.multiple_of((r >> 7) * LANES, LANES)
            pltpu.make_async_copy(
                embT_hbm.at[:, pl.ds(off, LANES)],
                bufs.at[g * _G + i],
                sems[g],
            ).start()
            bi = r // BCOLS
            bj = r - bi * BCOLS
            boff = pl.multiple_of((bj >> 7) * LANES, LANES)
            pltpu.make_async_copy(
                bias2_hbm.at[pl.ds(bi, 1), pl.ds(boff, LANES)],
                bbufs.at[g * _G + i],
                sems[g],
            ).start()

    def extract_batch(t):
        g = t % 2
        vec = idx_v[pl.ds((t * _G // 16) * 16, 16)]
        for i in range(_G):
            pltpu.make_async_copy(
                embT_hbm.at[:, pl.ds(0, LANES)], bufs.at[g * _G + i], sems[g]
            ).wait()
            pltpu.make_async_copy(
                bias2_hbm.at[pl.ds(0, 1), pl.ds(0, LANES)],
                bbufs.at[g * _G + i], sems[g]
            ).wait()
        for i in range(_G):
            k = t * _G + i
            r = vec[k % 16]
            m = r & (LANES - 1)
            slot = jnp.full((16,), g * _G + i, jnp.int32)
            mv = jnp.full((16,), m, jnp.int32)
            for cc in range(EMB // 16):
                vals = plsc.load_gather(bufs, [slot, iota16 + (16 * cc), mv])
                w_v[k, pl.ds(16 * cc, 16)] = vals
            bi = r // BCOLS
            bj = r - bi * BCOLS
            bm = bj & (LANES - 1)
            bvals = plsc.load_gather(
                bbufs, [slot, jnp.zeros((16,), jnp.int32),
                        jnp.full((16,), bm, jnp.int32)])
            ch = k // 16
            cur = bias_v[pl.ds(ch * 16, 16)]
            bias_v[pl.ds(ch * 16, 16)] = jnp.where(
                iota16 == (k % 16), bvals, cur)

    fire_batch(0)
    for t in range(1, _NT):
        fire_batch(t)
        extract_batch(t - 1)
    extract_batch(_NT - 1)

    pltpu.sync_copy(w_v, w_out.at[pl.ds(base, _BPW)])
    pltpu.sync_copy(bias_v, b_out.at[pl.ds(base, _BPW)])


_TM = 1024
_TN = 1024
_NI = B // _TM
_NJ = B // _TN


def _loss_body(w_i_ref, wT_j_ref, bcol_ref, brow_ref, lx_ref, wgt_ref, out_ref):
    i = pl.program_id(0)
    j = pl.program_id(1)
    t = jnp.dot(w_i_ref[...], wT_j_ref[...], preferred_element_type=jnp.float32)
    d = t + bcol_ref[...] + brow_ref[...] - lx_ref[...]
    s = jnp.sum(wgt_ref[...] * d * d).reshape(1, 1)

    is_first = (i == 0) & (j == 0)
    is_last = (i == _NI - 1) & (j == _NJ - 1)

    @pl.when(is_first)
    def _():
        out_ref[...] = s

    @pl.when(jnp.logical_not(is_first))
    def _():
        out_ref[...] = out_ref[...] + s

    @pl.when(is_last)
    def _():
        out_ref[...] = out_ref[...] * (1.0 / (B * B))


def _tc_loss(w, wT, b_col, b_row, log_x, weights):
    return pl.pallas_call(
        _loss_body,
        grid=(_NI, _NJ),
        in_specs=[
            pl.BlockSpec((_TM, EMB), lambda i, j: (i, 0)),
            pl.BlockSpec((EMB, _TN), lambda i, j: (0, j)),
            pl.BlockSpec((_TM, 1), lambda i, j: (i, 0)),
            pl.BlockSpec((1, _TN), lambda i, j: (0, j)),
            pl.BlockSpec((_TM, _TN), lambda i, j: (i, j)),
            pl.BlockSpec((_TM, _TN), lambda i, j: (i, j)),
        ],
        out_specs=pl.BlockSpec((1, 1), lambda i, j: (0, 0)),
        out_shape=jax.ShapeDtypeStruct((1, 1), jnp.float32),
    )(w, wT, b_col, b_row, log_x, weights)


def kernel(indices, log_x, weights, emb_table, bias_table):
    idx = indices.astype(jnp.int32)
    w, b = _sc_gather(emb_table.T, bias_table.reshape(BROWS, BCOLS), idx)
    loss = _tc_loss(w, w.T, b.reshape(B, 1), b.reshape(1, B), log_x, weights)
    return loss[0, 0]
